# Initial kernel scaffold; baseline (speedup 1.0000x reference)
import jax
import jax.numpy as jnp
from jax.experimental import pallas as pl


def kernel(x, Wg, W1, W2, W3, Ws1, Ws2, Ws3):
    raise NotImplementedError("write your pallas kernel here")


# speedup vs baseline: 2.3558x; 2.3558x over previous
import functools

import jax
import jax.numpy as jnp
from jax.experimental import pallas as pl
from jax.experimental.pallas import tpu as pltpu

E = 8
K = 2


def _moe_dense_kernel(x_ref, wg_ref, w1_ref, w3_ref, w2_ref, out_ref, comb_ref):
    e = pl.program_id(0)

    @pl.when(e == 0)
    def _routing():
        x = x_ref[...]
        logits = jax.lax.dot_general(
            x, wg_ref[...], (((1,), (1,)), ((), ())),
            preferred_element_type=jnp.float32)
        iota = jax.lax.broadcasted_iota(jnp.int32, logits.shape, 1)
        m1 = jnp.max(logits, axis=1, keepdims=True)
        i1 = jnp.min(jnp.where(logits == m1, iota, E), axis=1, keepdims=True)
        masked = jnp.where(iota == i1, -jnp.inf, logits)
        m2 = jnp.max(masked, axis=1, keepdims=True)
        i2 = jnp.min(jnp.where((logits == m2) & (iota != i1), iota, E),
                     axis=1, keepdims=True)
        e2 = jnp.exp(m2 - m1)
        denom = 1.0 + e2
        comb_ref[...] = (jnp.where(iota == i1, 1.0, 0.0)
                         + jnp.where(iota == i2, e2, 0.0)) / denom

    x = x_ref[...]
    g = jax.lax.dot_general(x, w1_ref[0], (((1,), (1,)), ((), ())),
                            preferred_element_type=jnp.float32)
    u = jax.lax.dot_general(x, w3_ref[0], (((1,), (1,)), ((), ())),
                            preferred_element_type=jnp.float32)
    h = (g * jax.lax.logistic(g)) * u
    comb = comb_ref[...]
    lane = jax.lax.broadcasted_iota(jnp.int32, comb.shape, 1)
    comb_col = jnp.sum(jnp.where(lane == e, comb, 0.0), axis=1, keepdims=True)
    h = h * comb_col
    contrib = jax.lax.dot_general(h, w2_ref[0], (((1,), (1,)), ((), ())),
                                  preferred_element_type=jnp.float32)

    @pl.when(e == 0)
    def _init():
        out_ref[...] = contrib

    @pl.when(e != 0)
    def _acc():
        out_ref[...] += contrib


def _moe_dense(xf, Wg, W1, W3, W2, interpret=False):
    T, H = xf.shape
    DFF = W1.shape[1]
    return pl.pallas_call(
        _moe_dense_kernel,
        grid=(E,),
        in_specs=[
            pl.BlockSpec((T, H), lambda e: (0, 0)),
            pl.BlockSpec((E, H), lambda e: (0, 0)),
            pl.BlockSpec((1, DFF, H), lambda e: (e, 0, 0)),
            pl.BlockSpec((1, DFF, H), lambda e: (e, 0, 0)),
            pl.BlockSpec((1, H, DFF), lambda e: (e, 0, 0)),
        ],
        out_specs=pl.BlockSpec((T, H), lambda e: (0, 0)),
        out_shape=jax.ShapeDtypeStruct((T, H), jnp.float32),
        scratch_shapes=[pltpu.VMEM((T, E), jnp.float32)],
        compiler_params=pltpu.CompilerParams(
            dimension_semantics=("arbitrary",)),
        interpret=interpret,
    )(xf, Wg, W1, W3, W2)


def kernel(x, Wg, W1, W2, W3, Ws1, Ws2, Ws3):
    B, S, H = x.shape
    xf = x.reshape(B * S, H)
    y = _moe_dense(xf, Wg, W1, W3, W2)
    return y.reshape(B, S, H)
